# baseline (speedup 1.0000x reference)
import functools

import jax
import jax.numpy as jnp
from jax import lax
from jax.experimental import pallas as pl
from jax.experimental.pallas import tpu as pltpu
from jax.experimental.pallas import tpu_sc as plsc

N_SEM = 8192
N_LRN = 128
N_CB = N_SEM + N_LRN
D = 1024
B = 16 * 1024
BT = 256
NI = B // BT
BETA = 0.25


def _vq_body(x_ref, cb_ref, logits_ref, idx_ref, vq_ref, com_ref, q_ref, acc_ref):
    i = pl.program_id(0)

    @pl.when(i == 0)
    def _():
        acc_ref[0] = 0.0
        acc_ref[1] = 0.0
        acc_ref[2] = 0.0

    logits = lax.dot_general(
        x_ref[...], cb_ref[...],
        dimension_numbers=(((1,), (1,)), ((), ())),
        preferred_element_type=jnp.float32,
    )
    logits_ref[...] = logits

    mx = jnp.max(logits, axis=1, keepdims=True)
    colsf = lax.broadcasted_iota(jnp.int32, (1, N_CB), 1).astype(jnp.float32)
    idxf = jnp.min(jnp.where(logits == mx, colsf, jnp.float32(2**30)),
                   axis=1, keepdims=True)
    idx = idxf.astype(jnp.int32)
    idx_ref[...] = idx

    lrn = (idx >= N_SEM).astype(jnp.float32)
    one_m = 1.0 - mx
    acc_ref[0] += jnp.sum(one_m)
    acc_ref[1] += jnp.sum(one_m * lrn)
    acc_ref[2] += jnp.sum(lrn)

    @pl.when(i == NI - 1)
    def _():
        com = acc_ref[0] / jnp.float32(B)
        vq = acc_ref[1] / (acc_ref[2] + 1e-6)
        com_ref[0, 0] = com
        vq_ref[0, 0] = vq
        q_ref[0, 0] = vq + BETA * com


_vq_call = pl.pallas_call(
    _vq_body,
    grid=(NI,),
    in_specs=[
        pl.BlockSpec((BT, D), lambda i: (i, 0)),
        pl.BlockSpec((N_CB, D), lambda i: (0, 0)),
    ],
    out_specs=[
        pl.BlockSpec((BT, N_CB), lambda i: (i, 0)),
        pl.BlockSpec((BT, 1), lambda i: (i, 0)),
        pl.BlockSpec((1, 1), lambda i: (0, 0), memory_space=pltpu.SMEM),
        pl.BlockSpec((1, 1), lambda i: (0, 0), memory_space=pltpu.SMEM),
        pl.BlockSpec((1, 1), lambda i: (0, 0), memory_space=pltpu.SMEM),
    ],
    out_shape=[
        jax.ShapeDtypeStruct((B, N_CB), jnp.float32),
        jax.ShapeDtypeStruct((B, 1), jnp.int32),
        jax.ShapeDtypeStruct((1, 1), jnp.float32),
        jax.ShapeDtypeStruct((1, 1), jnp.float32),
        jax.ShapeDtypeStruct((1, 1), jnp.float32),
    ],
    scratch_shapes=[pltpu.SMEM((4,), jnp.float32)],
)

_NC, _NS = 2, 16
_NW = _NC * _NS
_BPW = B // _NW
_CH = 32
_NCH = _BPW // _CH


def _gather_body(cb_hbm, idx_hbm, out_hbm, out2_hbm, idx_v, rows_v, sem_a, sem_b):
    wid = lax.axis_index("s") * _NC + lax.axis_index("c")
    base = wid * _BPW
    pltpu.sync_copy(idx_hbm.at[pl.ds(base, _BPW)], idx_v)
    sems = (sem_a, sem_b)

    def issue(c, buf):
        return pltpu.async_copy(
            cb_hbm.at[idx_v.at[pl.ds(c * _CH, _CH)]], rows_v.at[buf], sems[buf])

    copies = [issue(0, 0), None]
    for c in range(_NCH):
        buf = c & 1
        if c + 1 < _NCH:
            copies[1 - buf] = issue(c + 1, 1 - buf)
        copies[buf].wait()
        pltpu.sync_copy(rows_v.at[buf], out_hbm.at[pl.ds(base + c * _CH, _CH)])
        pltpu.sync_copy(rows_v.at[buf], out2_hbm.at[pl.ds(base + c * _CH, _CH)])


@functools.cache
def _gather_call():
    mesh = plsc.VectorSubcoreMesh(core_axis_name="c", subcore_axis_name="s")
    return pl.kernel(
        _gather_body,
        mesh=mesh,
        out_type=[jax.ShapeDtypeStruct((B, D), jnp.float32),
                  jax.ShapeDtypeStruct((B, D), jnp.float32)],
        scratch_types=[
            pltpu.VMEM((_BPW,), jnp.int32),
            pltpu.VMEM((2, _CH, D), jnp.float32),
            pltpu.SemaphoreType.DMA,
            pltpu.SemaphoreType.DMA,
        ],
    )


def _l2n(a, eps=1e-12):
    n = jnp.maximum(jnp.linalg.norm(a, axis=-1, keepdims=True), eps)
    return a / n


def kernel(x, semantic_embeddings, learnable_entries):
    xn = _l2n(x).reshape(B, D)
    cbn = jnp.concatenate(
        [_l2n(semantic_embeddings), _l2n(learnable_entries)], axis=0)
    logits, idx, vq, com, q = _vq_call(xn, cbn)
    idx_flat = idx.reshape(B)
    zq, zq_st = _gather_call()(cbn, idx_flat)
    return (
        logits.reshape(16, 1024, N_CB),
        idx_flat.reshape(16, 1024),
        zq.reshape(16, 1024, D),
        zq_st.reshape(16, 1024, D),
        vq.reshape(()),
        com.reshape(()),
        q.reshape(()),
    )
